# baseline (speedup 1.0000x reference)
import functools

import jax
import jax.numpy as jnp
from jax import lax
from jax.experimental import pallas as pl
from jax.experimental.pallas import tpu as pltpu
from jax.experimental.pallas import tpu_sc as plsc

B = 4
L_REMAIN = 2048
D = 1024
N = 8192
LFULL = N + 1
MASK_ROW = B * L_REMAIN
IR = 8
CRW = B * IR
QN = N // IR
NC, NS = 2, 16
NW = NC * NS
CPW = QN // NW
GA = CRW


def _sc_revert(table, ridx_flat, pos_enc, pm_flat):
    mesh = plsc.VectorSubcoreMesh(core_axis_name="c", subcore_axis_name="s")

    @functools.partial(
        pl.kernel,
        out_type=jax.ShapeDtypeStruct((B, LFULL, D), jnp.float32),
        mesh=mesh,
        compiler_params=pltpu.CompilerParams(needs_layout_passes=False),
        scratch_types=[
            pltpu.VMEM((B * L_REMAIN,), jnp.int32),
            pltpu.VMEM((B * 16,), jnp.int32),
            pltpu.VMEM((B * 16,), jnp.int32),
            pltpu.VMEM((48,), jnp.int32),
            pltpu.VMEM((48,), jnp.int32),
            pltpu.VMEM((CRW,), jnp.int32),
            pltpu.VMEM((CRW,), jnp.int32),
            pltpu.VMEM((GA, D), jnp.float32),
            pltpu.VMEM((GA, D), jnp.float32),
            pltpu.VMEM((IR, D), jnp.float32),
            pltpu.VMEM((IR, D), jnp.float32),
            pltpu.VMEM((CRW, D), jnp.float32),
            pltpu.SemaphoreType.DMA,
            pltpu.SemaphoreType.DMA,
            pltpu.SemaphoreType.DMA,
            pltpu.SemaphoreType.DMA,
            pltpu.SemaphoreType.DMA,
            pltpu.SemaphoreType.DMA,
            pltpu.SemaphoreType.DMA,
            pltpu.SemaphoreType.DMA,
        ],
    )
    def k(table_hbm, ridx_hbm, pos_hbm, pm_hbm, out_hbm,
          pm_v, rscr0, rscr1, cidx0, cidx1, rsrc0, rsrc1, ga0, ga1,
          pos0, pos1, wbuf,
          rsem0, rsem1, gsem0, gsem1, g2sem0, g2sem1, psem0, psem1):
        rscr_s = (rscr0, rscr1)
        cidx_s = (cidx0, cidx1)
        rsrc_s = (rsrc0, rsrc1)
        ga_s = (ga0, ga1)
        pos_s = (pos0, pos1)
        rsem_s = (rsem0, rsem1)
        gsem_s = (gsem0, gsem1)
        g2sem_s = (g2sem0, g2sem1)
        psem_s = (psem0, psem1)

        wid = lax.axis_index("s") * NC + lax.axis_index("c")
        pltpu.sync_copy(pm_hbm, pm_v)
        lanes = lax.iota(jnp.int32, 16)

        def src_and_mask(h, i0, d0, rscr):
            flat = lanes + 16 * h
            bv = flat // IR
            il = flat - bv * IR
            ivec = i0 + il
            scr_idx = bv * 16 + jnp.clip(il + d0 - 1, 0, 15)
            r = plsc.load_gather(rscr, [scr_idx])
            j = r + 1
            in_rng = (ivec > 0) & (j <= L_REMAIN - 1)
            fp = jnp.clip(bv * L_REMAIN + j - 1, 0, B * L_REMAIN - 1)
            pmv = plsc.load_gather(pm_v, [fp])
            valid = in_rng & (pmv == 1)
            src = jnp.where(valid, bv * L_REMAIN + j, MASK_ROW)
            return src, valid

        def start(n, s):
            q = wid * CPW + n
            i0 = pl.multiple_of(q * IR, IR)
            a0 = pl.multiple_of(jnp.maximum(i0 - 8, 0), 8)
            d0 = i0 - a0
            for b in range(B):
                pltpu.async_copy(ridx_hbm.at[pl.ds(b * N + a0, 16)],
                                 rscr_s[s].at[pl.ds(16 * b, 16)], rsem_s[s])
            for b in range(B):
                pltpu.make_async_copy(ridx_hbm.at[pl.ds(b * N + a0, 16)],
                                      rscr_s[s].at[pl.ds(16 * b, 16)],
                                      rsem_s[s]).wait()
            src0, m0 = src_and_mask(0, i0, d0, rscr_s[s])
            src1, m1 = src_and_mask(1, i0, d0, rscr_s[s])
            m0i = m0.astype(jnp.int32)
            m1i = m1.astype(jnp.int32)
            v0 = jnp.sum(m0i)
            v1 = jnp.sum(m1i)
            v = v0 + v1
            plsc.store_compressed(cidx_s[s].at[pl.ds(0, 16)], src0, mask=m0)
            plsc.store_compressed(cidx_s[s].at[pl.ds(v0, 16)], src1, mask=m1)
            cidx_s[s][pl.ds(v, 16)] = jnp.full((16,), MASK_ROW, jnp.int32)
            pc0 = plsc.cumsum(m0i) - m0i
            pc1 = v0 + plsc.cumsum(m1i) - m1i
            rsrc_s[s][pl.ds(0, 16)] = jnp.where(m0, pc0, v)
            rsrc_s[s][pl.ds(16, 16)] = jnp.where(m1, pc1, v)
            pltpu.async_copy(table_hbm.at[cidx_s[s].at[pl.ds(0, 16)]],
                             ga_s[s].at[pl.ds(0, 16), :], gsem_s[s])

            @pl.when(v >= 16)
            def _():
                pltpu.async_copy(table_hbm.at[cidx_s[s].at[pl.ds(16, 16)]],
                                 ga_s[s].at[pl.ds(16, 16), :], g2sem_s[s])

            pltpu.async_copy(pos_hbm.at[pl.ds(i0, IR), :], pos_s[s], psem_s[s])
            return v

        def finish(n, s, v):
            q = wid * CPW + n
            i0 = pl.multiple_of(q * IR, IR)
            pltpu.make_async_copy(table_hbm.at[cidx_s[s].at[pl.ds(0, 16)]],
                                  ga_s[s].at[pl.ds(0, 16), :], gsem_s[s]).wait()

            @pl.when(v >= 16)
            def _():
                pltpu.make_async_copy(table_hbm.at[cidx_s[s].at[pl.ds(16, 16)]],
                                      ga_s[s].at[pl.ds(16, 16), :],
                                      g2sem_s[s]).wait()

            pltpu.make_async_copy(pos_hbm.at[pl.ds(i0, IR), :], pos_s[s],
                                  psem_s[s]).wait()

            def row_body(rr, carry2):
                slot = plsc.load_gather(rsrc_s[s], [jnp.full((16,), rr, jnp.int32)])
                il = rr - (rr // IR) * IR
                for kk in range(D // 16):
                    col = kk * 16 + lanes
                    g = plsc.load_gather(ga_s[s], [slot, col])
                    wbuf[rr, pl.ds(kk * 16, 16)] = g + pos_s[s][il, pl.ds(kk * 16, 16)]
                return carry2

            lax.fori_loop(0, CRW, row_body, 0)
            for b in range(B):
                pltpu.sync_copy(wbuf.at[pl.ds(IR * b, IR), :],
                                out_hbm.at[b, pl.ds(i0, IR), :])

        va = start(0, 0)
        vb = start(1, 1)

        def body(g, carry):
            va, vb = carry
            finish(2 * g, 0, va)
            va2 = start(2 * g + 2, 0)
            finish(2 * g + 1, 1, vb)
            vb2 = start(2 * g + 3, 1)
            return (va2, vb2)

        va, vb = lax.fori_loop(0, CPW // 2 - 1, body, (va, vb))
        finish(CPW - 2, 0, va)
        finish(CPW - 1, 1, vb)

        @pl.when(wid < B)
        def _():
            bt = wid
            pltpu.sync_copy(ridx_hbm.at[pl.ds(bt * N + N - 16, 16)],
                            rscr0.at[pl.ds(0, 16)])
            r = plsc.load_gather(rscr0, [jnp.full((16,), 15, jnp.int32)])
            j = r + 1
            in_rng = j <= L_REMAIN - 1
            fp = jnp.clip(bt * L_REMAIN + j - 1, 0, B * L_REMAIN - 1)
            pmv = plsc.load_gather(pm_v, [fp])
            valid = in_rng & (pmv == 1)
            src = jnp.where(valid, bt * L_REMAIN + j, MASK_ROW)
            cidx0[pl.ds(0, 16)] = src
            pltpu.async_copy(table_hbm.at[cidx0.at[pl.ds(0, 16)]],
                             ga0.at[pl.ds(0, 16), :], gsem0).wait()
            pltpu.async_copy(pos_hbm.at[pl.ds(N, IR), :], pos0, psem0).wait()
            for kk in range(D // 16):
                sl = pl.ds(kk * 16, 16)
                wbuf[0, sl] = ga0[0, sl] + pos0[0, sl]
            pltpu.sync_copy(wbuf.at[pl.ds(0, 1), :],
                            out_hbm.at[bt, pl.ds(N, 1), :])

    return k(table, ridx_flat, pos_enc, pm_flat)


def kernel(temporal_data, revert_idx, temporal_pos_enc, remain_padding_mask, mask_token):
    table = jnp.concatenate(
        [temporal_data.reshape(B * L_REMAIN, D), mask_token], axis=0)
    ridx_flat = revert_idx.reshape(B * N)
    pm_flat = jnp.pad(remain_padding_mask, ((0, 0), (0, 1))).reshape(B * L_REMAIN)
    return _sc_revert(table, ridx_flat, temporal_pos_enc, pm_flat)
